# baseline (speedup 1.0000x reference)
import jax
import jax.numpy as jnp
from jax import lax
from jax.experimental import pallas as pl
from jax.experimental.pallas import tpu as pltpu
from jax.experimental.pallas import tpu_sc as plsc

N_NODES = 10000
N_EDGES = 320000
EDGE_DIM = 16
NODE_DIM = 128

NC = 2
NS = 16
NW = NC * NS
EPW = N_EDGES // NW
BATCH = 80
NB = EPW // BATCH
CHUNK = 2000
N_CHUNKS = EPW // CHUNK
BPC = CHUNK // BATCH
ROWS_PER_TILE = N_NODES // NS


def _sc_scatter_body(ei_hbm, eattr_hbm, out_sums, out_cnts,
                     ebuf128, ebuf, ibuf1, ibuf2, ones_v, z2d,
                     sums_sh, cnts_sh):
    c = lax.axis_index("c")
    s = lax.axis_index("s")
    w = c * NS + s

    z16 = jnp.zeros((16,), jnp.float32)
    o16 = jnp.ones((16,), jnp.float32)

    def fill_z(i, carry):
        z2d[i, :] = z16
        return carry

    lax.fori_loop(0, ROWS_PER_TILE, fill_z, 0)

    def fill_o(i, carry):
        ones_v[i, :] = o16
        return carry

    lax.fori_loop(0, BATCH, fill_o, 0)

    pltpu.sync_copy(ei_hbm.at[0, pl.ds(w * EPW, EPW)], ibuf1)

    def repack_i(i, carry):
        v = ibuf1[pl.ds(i * 16, 16)]
        ibuf2[i // (BATCH // 16), pl.ds((i % (BATCH // 16)) * 16, 16)] = v
        return carry

    lax.fori_loop(0, EPW // 16, repack_i, 0)

    pltpu.sync_copy(z2d, sums_sh.at[pl.ds(s * ROWS_PER_TILE, ROWS_PER_TILE)])
    pltpu.sync_copy(z2d, cnts_sh.at[pl.ds(s * ROWS_PER_TILE, ROWS_PER_TILE)])
    plsc.subcore_barrier()

    for k in range(N_CHUNKS):
        rbase = (w * EPW + k * CHUNK) // 8
        pltpu.sync_copy(eattr_hbm.at[pl.ds(rbase, CHUNK // 8)], ebuf128)

        def repack_e(r, carry):
            for j in range(8):
                ebuf[r * 8 + j, :] = ebuf128[r, pl.ds(j * 16, 16)]
            return carry

        lax.fori_loop(0, CHUNK // 8, repack_e, 0)

        def scat(b, carry):
            idx = ibuf2.at[k * BPC + b]
            pltpu.sync_copy(ebuf.at[pl.ds(b * BATCH, BATCH)],
                            sums_sh.at[idx], add=True)
            pltpu.sync_copy(ones_v, cnts_sh.at[idx], add=True)
            return carry

        lax.fori_loop(0, BPC, scat, 0)

    plsc.subcore_barrier()

    row0 = s * ROWS_PER_TILE
    pltpu.sync_copy(sums_sh.at[pl.ds(row0, ROWS_PER_TILE)], out_sums.at[c, s])
    pltpu.sync_copy(cnts_sh.at[pl.ds(row0, ROWS_PER_TILE)], out_cnts.at[c, s])


@jax.jit
def _sc_scatter(edge_index, edge_attr128):
    mesh = plsc.VectorSubcoreMesh(core_axis_name="c", subcore_axis_name="s")
    f = pl.kernel(
        _sc_scatter_body,
        out_type=(
            jax.ShapeDtypeStruct((NC, NS, ROWS_PER_TILE, EDGE_DIM), jnp.float32),
            jax.ShapeDtypeStruct((NC, NS, ROWS_PER_TILE, EDGE_DIM), jnp.float32),
        ),
        mesh=mesh,
        compiler_params=pltpu.CompilerParams(use_tc_tiling_on_sc=False),
        scratch_types=[
            pltpu.VMEM((CHUNK // 8, 128), jnp.float32),
            pltpu.VMEM((CHUNK, EDGE_DIM), jnp.float32),
            pltpu.VMEM((EPW,), jnp.int32),
            pltpu.VMEM((NB, BATCH), jnp.int32),
            pltpu.VMEM((BATCH, EDGE_DIM), jnp.float32),
            pltpu.VMEM((ROWS_PER_TILE, EDGE_DIM), jnp.float32),
            pltpu.VMEM_SHARED((N_NODES, EDGE_DIM), jnp.float32),
            pltpu.VMEM_SHARED((N_NODES, EDGE_DIM), jnp.float32),
        ],
    )
    return f(edge_index, edge_attr128)


def _tc_mlp_body(x_ref, sums_ref, cnts_ref, w1a_ref, w1b_ref, b1_ref,
                 w2_ref, b2_ref, w3_ref, b3_ref, wo_ref, bo_ref,
                 g_ref, bt_ref, out_ref):
    sums = sums_ref[0] + sums_ref[1]
    cnt = cnts_ref[0, :, 0:1] + cnts_ref[1, :, 0:1]
    agg = sums / jnp.maximum(cnt, 1.0)

    g = g_ref[...]
    bt = bt_ref[...]

    h = (jnp.dot(x_ref[...], w1a_ref[...], preferred_element_type=jnp.float32)
         + jnp.dot(agg, w1b_ref[...], preferred_element_type=jnp.float32)
         + b1_ref[...])

    for w_ref, b_ref in ((w2_ref, b2_ref), (w3_ref, b3_ref), (None, None)):
        h = jnp.maximum(h, 0.0)
        mu = jnp.mean(h, axis=0, keepdims=True)
        d = h - mu
        var = jnp.mean(d * d, axis=0, keepdims=True)
        h = g * d / jnp.sqrt(var + 1e-5) + bt
        if w_ref is not None:
            h = jnp.dot(h, w_ref[...], preferred_element_type=jnp.float32) + b_ref[...]

    out_ref[...] = (jnp.dot(h, wo_ref[...], preferred_element_type=jnp.float32)
                    + bo_ref[...])


@jax.jit
def _tc_mlp(x, sums, cnts, w1a, w1b, b1, w2, b2, w3, b3, wo, bo, g, bt):
    return pl.pallas_call(
        _tc_mlp_body,
        out_shape=jax.ShapeDtypeStruct((N_NODES, 64), jnp.float32),
    )(x, sums, cnts, w1a, w1b, b1, w2, b2, w3, b3, wo, bo, g, bt)


def kernel(x, edge_index, edge_attr, W1, b1, W2, b2, W3, b3, Wout, bout,
           gamma, beta):
    sums, cnts = _sc_scatter(edge_index.astype(jnp.int32),
                             edge_attr.reshape(N_EDGES // 8, 8 * EDGE_DIM))
    sums = sums.reshape(NC, N_NODES, EDGE_DIM)
    cnts = cnts.reshape(NC, N_NODES, EDGE_DIM)
    r = lambda v: v.reshape(1, -1)
    return _tc_mlp(x, sums, cnts, W1[:NODE_DIM], W1[NODE_DIM:], r(b1),
                   W2, r(b2), W3, r(b3), Wout, r(bout), r(gamma), r(beta))


# speedup vs baseline: 1.0499x; 1.0499x over previous
import jax
import jax.numpy as jnp
from jax import lax
from jax.experimental import pallas as pl
from jax.experimental.pallas import tpu as pltpu
from jax.experimental.pallas import tpu_sc as plsc

N_NODES = 10000
N_EDGES = 320000
EDGE_DIM = 16
NODE_DIM = 128

NC = 2
NS = 16
NW = NC * NS
BATCH = 128
NPAD = 128
ROWS = N_EDGES // BATCH
ROWS_PAD = NW * 80
RPW = ROWS_PAD // NW
EPW = RPW * BATCH
CHUNK_R = 20
N_CHUNKS = RPW // CHUNK_R
CHUNK_E = CHUNK_R * BATCH
ACC_ROWS = N_NODES + NPAD
ZPT = ACC_ROWS // NS
ROWS_PER_TILE = N_NODES // NS
EA_RPW = EPW * EDGE_DIM // 128
EA_RPC = CHUNK_E * EDGE_DIM // 128


def _zero_acc(z2d, acc, s):
    z16 = jnp.zeros((16,), jnp.float32)

    def fill_z(i, carry):
        z2d[i, :] = z16
        return carry

    lax.fori_loop(0, ZPT, fill_z, 0)
    pltpu.sync_copy(z2d, acc.at[pl.ds(s * ZPT, ZPT)])


def _sc_counts_body(idx_hbm, out_cnts, ibuf, ones_v, z2d, cnts_sh):
    c = lax.axis_index("c")
    s = lax.axis_index("s")
    w = c * NS + s

    o16 = jnp.ones((16,), jnp.float32)

    def fill_o(i, carry):
        ones_v[i, :] = o16
        return carry

    lax.fori_loop(0, BATCH, fill_o, 0)

    _zero_acc(z2d, cnts_sh, s)
    pltpu.sync_copy(idx_hbm.at[pl.ds(w * RPW, RPW)], ibuf)
    plsc.subcore_barrier()

    def scat(b, carry):
        pltpu.sync_copy(ones_v, cnts_sh.at[ibuf.at[b]], add=True)
        return carry

    lax.fori_loop(0, RPW, scat, 0)
    plsc.subcore_barrier()

    pltpu.sync_copy(cnts_sh.at[pl.ds(s * ROWS_PER_TILE, ROWS_PER_TILE)],
                    out_cnts.at[c, s])


def _sc_sums_body(idx_hbm, eattr_hbm, out_sums, ibuf, ebuf128, ebuf, z2d,
                  sums_sh):
    c = lax.axis_index("c")
    s = lax.axis_index("s")
    w = c * NS + s

    _zero_acc(z2d, sums_sh, s)
    pltpu.sync_copy(idx_hbm.at[pl.ds(w * RPW, RPW)], ibuf)
    plsc.subcore_barrier()

    for k in range(N_CHUNKS):
        rbase = w * EA_RPW + k * EA_RPC
        pltpu.sync_copy(eattr_hbm.at[pl.ds(rbase, EA_RPC)], ebuf128)

        def repack_e(r, carry):
            for j in range(8):
                ebuf[r * 8 + j, :] = ebuf128[r, pl.ds(j * 16, 16)]
            return carry

        lax.fori_loop(0, EA_RPC, repack_e, 0)

        def scat(b, carry):
            pltpu.sync_copy(ebuf.at[pl.ds(b * BATCH, BATCH)],
                            sums_sh.at[ibuf.at[k * CHUNK_R + b]], add=True)
            return carry

        lax.fori_loop(0, CHUNK_R, scat, 0)

    plsc.subcore_barrier()

    pltpu.sync_copy(sums_sh.at[pl.ds(s * ROWS_PER_TILE, ROWS_PER_TILE)],
                    out_sums.at[c, s])


_MESH = dict(core_axis_name="c", subcore_axis_name="s")
_PARTIAL = jax.ShapeDtypeStruct((NC, NS, ROWS_PER_TILE, EDGE_DIM), jnp.float32)
_PARAMS = pltpu.CompilerParams(use_tc_tiling_on_sc=False)


@jax.jit
def _sc_counts(idx_pad):
    f = pl.kernel(
        _sc_counts_body,
        out_type=_PARTIAL,
        mesh=plsc.VectorSubcoreMesh(**_MESH),
        compiler_params=_PARAMS,
        scratch_types=[
            pltpu.VMEM((RPW, BATCH), jnp.int32),
            pltpu.VMEM((BATCH, EDGE_DIM), jnp.float32),
            pltpu.VMEM((ZPT, EDGE_DIM), jnp.float32),
            pltpu.VMEM_SHARED((ACC_ROWS, EDGE_DIM), jnp.float32),
        ],
    )
    return f(idx_pad)


@jax.jit
def _sc_sums(idx_pad, eattr128):
    f = pl.kernel(
        _sc_sums_body,
        out_type=_PARTIAL,
        mesh=plsc.VectorSubcoreMesh(**_MESH),
        compiler_params=_PARAMS,
        scratch_types=[
            pltpu.VMEM((RPW, BATCH), jnp.int32),
            pltpu.VMEM((EA_RPC, 128), jnp.float32),
            pltpu.VMEM((CHUNK_E, EDGE_DIM), jnp.float32),
            pltpu.VMEM((ZPT, EDGE_DIM), jnp.float32),
            pltpu.VMEM_SHARED((ACC_ROWS, EDGE_DIM), jnp.float32),
        ],
    )
    return f(idx_pad, eattr128)


def _tc_mlp_body(x_ref, sums_ref, cnts_ref, w1a_ref, w1b_ref, b1_ref,
                 w2_ref, b2_ref, w3_ref, b3_ref, wo_ref, bo_ref,
                 g_ref, bt_ref, out_ref):
    sums = sums_ref[0] + sums_ref[1]
    cnt = cnts_ref[0, :, 0:1] + cnts_ref[1, :, 0:1]
    agg = sums / jnp.maximum(cnt, 1.0)

    g = g_ref[...]
    bt = bt_ref[...]

    h = (jnp.dot(x_ref[...], w1a_ref[...], preferred_element_type=jnp.float32)
         + jnp.dot(agg, w1b_ref[...], preferred_element_type=jnp.float32)
         + b1_ref[...])

    for w_ref, b_ref in ((w2_ref, b2_ref), (w3_ref, b3_ref), (None, None)):
        h = jnp.maximum(h, 0.0)
        mu = jnp.mean(h, axis=0, keepdims=True)
        d = h - mu
        var = jnp.mean(d * d, axis=0, keepdims=True)
        h = g * d / jnp.sqrt(var + 1e-5) + bt
        if w_ref is not None:
            h = jnp.dot(h, w_ref[...], preferred_element_type=jnp.float32) + b_ref[...]

    out_ref[...] = (jnp.dot(h, wo_ref[...], preferred_element_type=jnp.float32)
                    + bo_ref[...])


@jax.jit
def _tc_mlp(x, sums, cnts, w1a, w1b, b1, w2, b2, w3, b3, wo, bo, g, bt):
    return pl.pallas_call(
        _tc_mlp_body,
        out_shape=jax.ShapeDtypeStruct((N_NODES, 64), jnp.float32),
    )(x, sums, cnts, w1a, w1b, b1, w2, b2, w3, b3, wo, bo, g, bt)


def kernel(x, edge_index, edge_attr, W1, b1, W2, b2, W3, b3, Wout, bout,
           gamma, beta):
    n_fake = ROWS_PAD * BATCH - N_EDGES
    sentinel = (N_NODES
                + jnp.arange(n_fake, dtype=jnp.int32) % NPAD)
    idx_pad = jnp.concatenate(
        [edge_index[0].astype(jnp.int32), sentinel]).reshape(ROWS_PAD, BATCH)
    cnts = _sc_counts(idx_pad)

    ea128 = jnp.pad(edge_attr.reshape(N_EDGES // 8, 8 * EDGE_DIM),
                    ((0, (ROWS_PAD * BATCH - N_EDGES) * EDGE_DIM // 128), (0, 0)))
    sums = _sc_sums(idx_pad, ea128)

    sums = sums.reshape(NC, N_NODES, EDGE_DIM)
    cnts = cnts.reshape(NC, N_NODES, EDGE_DIM)
    r = lambda v: v.reshape(1, -1)
    return _tc_mlp(x, sums, cnts, W1[:NODE_DIM], W1[NODE_DIM:], r(b1),
                   W2, r(b2), W3, r(b3), Wout, r(bout), r(gamma), r(beta))


# speedup vs baseline: 1.1214x; 1.0681x over previous
import jax
import jax.numpy as jnp
from jax import lax
from jax.experimental import pallas as pl
from jax.experimental.pallas import tpu as pltpu
from jax.experimental.pallas import tpu_sc as plsc

N_NODES = 10000
N_EDGES = 320000
EDGE_DIM = 16
NODE_DIM = 128

NC = 2
NS = 16
NW = NC * NS
BATCH = 128
NPAD = 128
ROWS = N_EDGES // BATCH
ROWS_PAD = NW * 80
RPW = ROWS_PAD // NW
EPW = RPW * BATCH
CHUNK_R = 20
N_CHUNKS = RPW // CHUNK_R
CHUNK_E = CHUNK_R * BATCH
ACC_ROWS = N_NODES + NPAD
ZPT = ACC_ROWS // NS
ROWS_PER_TILE = N_NODES // NS
EA_RPW = EPW * EDGE_DIM // 128
EA_RPC = CHUNK_E * EDGE_DIM // 128


def _zero_acc(z2d, acc, s):
    z16 = jnp.zeros((16,), jnp.float32)

    def fill_z(i, carry):
        z2d[i, :] = z16
        return carry

    lax.fori_loop(0, ZPT, fill_z, 0)
    pltpu.sync_copy(z2d, acc.at[pl.ds(s * ZPT, ZPT)])


def _sc_counts_body(idx_hbm, out_cnts, ibuf, ones_v, z2d, cnts_sh):
    c = lax.axis_index("c")
    s = lax.axis_index("s")
    w = c * NS + s

    o16 = jnp.ones((16,), jnp.float32)

    def fill_o(i, carry):
        ones_v[i, :] = o16
        return carry

    lax.fori_loop(0, BATCH, fill_o, 0)

    _zero_acc(z2d, cnts_sh, s)
    pltpu.sync_copy(idx_hbm.at[pl.ds(w * RPW, RPW)], ibuf)
    plsc.subcore_barrier()

    def scat(b, carry):
        pltpu.sync_copy(ones_v, cnts_sh.at[ibuf.at[b]], add=True)
        return carry

    lax.fori_loop(0, RPW, scat, 0)
    plsc.subcore_barrier()

    pltpu.sync_copy(cnts_sh.at[pl.ds(s * ROWS_PER_TILE, ROWS_PER_TILE)],
                    out_cnts.at[c, s])


def _sc_sums_body(idx_hbm, eattr_hbm, out_sums, ibuf, ebuf128, ebuf, z2d,
                  sums_sh):
    c = lax.axis_index("c")
    s = lax.axis_index("s")
    w = c * NS + s

    _zero_acc(z2d, sums_sh, s)
    pltpu.sync_copy(idx_hbm.at[pl.ds(w * RPW, RPW)], ibuf)
    plsc.subcore_barrier()

    max_rbase = N_EDGES * EDGE_DIM // 128 - EA_RPC
    for k in range(N_CHUNKS):
        rbase = jnp.minimum(w * EA_RPW + k * EA_RPC, max_rbase)
        pltpu.sync_copy(eattr_hbm.at[pl.ds(rbase, EA_RPC)], ebuf128)

        def repack_e(r, carry):
            for j in range(8):
                ebuf[r * 8 + j, :] = ebuf128[r, pl.ds(j * 16, 16)]
            return carry

        lax.fori_loop(0, EA_RPC, repack_e, 0)

        def scat(b, carry):
            pltpu.sync_copy(ebuf.at[pl.ds(b * BATCH, BATCH)],
                            sums_sh.at[ibuf.at[k * CHUNK_R + b]], add=True)
            return carry

        lax.fori_loop(0, CHUNK_R, scat, 0)

    plsc.subcore_barrier()

    pltpu.sync_copy(sums_sh.at[pl.ds(s * ROWS_PER_TILE, ROWS_PER_TILE)],
                    out_sums.at[c, s])


_MESH = dict(core_axis_name="c", subcore_axis_name="s")
_PARTIAL = jax.ShapeDtypeStruct((NC, NS, ROWS_PER_TILE, EDGE_DIM), jnp.float32)
_PARAMS = pltpu.CompilerParams(use_tc_tiling_on_sc=False)


@jax.jit
def _sc_counts(idx_pad):
    f = pl.kernel(
        _sc_counts_body,
        out_type=_PARTIAL,
        mesh=plsc.VectorSubcoreMesh(**_MESH),
        compiler_params=_PARAMS,
        scratch_types=[
            pltpu.VMEM((RPW, BATCH), jnp.int32),
            pltpu.VMEM((BATCH, EDGE_DIM), jnp.float32),
            pltpu.VMEM((ZPT, EDGE_DIM), jnp.float32),
            pltpu.VMEM_SHARED((ACC_ROWS, EDGE_DIM), jnp.float32),
        ],
    )
    return f(idx_pad)


@jax.jit
def _sc_sums(idx_pad, eattr128):
    f = pl.kernel(
        _sc_sums_body,
        out_type=_PARTIAL,
        mesh=plsc.VectorSubcoreMesh(**_MESH),
        compiler_params=_PARAMS,
        scratch_types=[
            pltpu.VMEM((RPW, BATCH), jnp.int32),
            pltpu.VMEM((EA_RPC, 128), jnp.float32),
            pltpu.VMEM((CHUNK_E, EDGE_DIM), jnp.float32),
            pltpu.VMEM((ZPT, EDGE_DIM), jnp.float32),
            pltpu.VMEM_SHARED((ACC_ROWS, EDGE_DIM), jnp.float32),
        ],
    )
    return f(idx_pad, eattr128)


def _tc_mlp_body(x_ref, sums_ref, cnts_ref, w1a_ref, w1b_ref, b1_ref,
                 w2_ref, b2_ref, w3_ref, b3_ref, wo_ref, bo_ref,
                 g_ref, bt_ref, out_ref):
    sums = sums_ref[0] + sums_ref[1]
    cnt = cnts_ref[0, :, 0:1] + cnts_ref[1, :, 0:1]
    agg = sums / jnp.maximum(cnt, 1.0)

    g = g_ref[...]
    bt = bt_ref[...]

    h = (jnp.dot(x_ref[...], w1a_ref[...], preferred_element_type=jnp.float32)
         + jnp.dot(agg, w1b_ref[...], preferred_element_type=jnp.float32)
         + b1_ref[...])

    for w_ref, b_ref in ((w2_ref, b2_ref), (w3_ref, b3_ref), (None, None)):
        h = jnp.maximum(h, 0.0)
        mu = jnp.mean(h, axis=0, keepdims=True)
        d = h - mu
        var = jnp.mean(d * d, axis=0, keepdims=True)
        h = g * d / jnp.sqrt(var + 1e-5) + bt
        if w_ref is not None:
            h = jnp.dot(h, w_ref[...], preferred_element_type=jnp.float32) + b_ref[...]

    out_ref[...] = (jnp.dot(h, wo_ref[...], preferred_element_type=jnp.float32)
                    + bo_ref[...])


@jax.jit
def _tc_mlp(x, sums, cnts, w1a, w1b, b1, w2, b2, w3, b3, wo, bo, g, bt):
    return pl.pallas_call(
        _tc_mlp_body,
        out_shape=jax.ShapeDtypeStruct((N_NODES, 64), jnp.float32),
    )(x, sums, cnts, w1a, w1b, b1, w2, b2, w3, b3, wo, bo, g, bt)


def kernel(x, edge_index, edge_attr, W1, b1, W2, b2, W3, b3, Wout, bout,
           gamma, beta):
    n_fake = ROWS_PAD * BATCH - N_EDGES
    sentinel = (N_NODES
                + jnp.arange(n_fake, dtype=jnp.int32) % NPAD)
    idx_pad = jnp.concatenate(
        [edge_index[0].astype(jnp.int32), sentinel]).reshape(ROWS_PAD, BATCH)
    cnts = _sc_counts(idx_pad)

    ea128 = edge_attr.reshape(N_EDGES // 8, 8 * EDGE_DIM)
    sums = _sc_sums(idx_pad, ea128)

    sums = sums.reshape(NC, N_NODES, EDGE_DIM)
    cnts = cnts.reshape(NC, N_NODES, EDGE_DIM)
    r = lambda v: v.reshape(1, -1)
    return _tc_mlp(x, sums, cnts, W1[:NODE_DIM], W1[NODE_DIM:], r(b1),
                   W2, r(b2), W3, r(b3), Wout, r(bout), r(gamma), r(beta))
